# Initial kernel scaffold; baseline (speedup 1.0000x reference)
import functools

import jax
import jax.numpy as jnp
from jax import lax
from jax.experimental import pallas as pl
from jax.experimental.pallas import tpu as pltpu
from jax.experimental.pallas import tpu_sc as plsc

N = 100000
E = 1600000
NP = 100096
RPS = NP // 16
ZR = 782
EPAD = 1638400
PAD = EPAD - E
SUP1 = 4096
NSUP1 = 25
SUP2 = 2048
NSUP2 = 25

_mesh = plsc.VectorSubcoreMesh(core_axis_name="c", subcore_axis_name="s")


def _scale_chunk(rows_v, w_v, wbase, idx_consts):
    @pl.loop(0, 8)
    def _(g):
        wv = w_v[pl.ds(wbase + g * 16, 16)]
        for e in range(16):
            r = g * 16 + e
            splat = wv.at[idx_consts[e]].get(mode="promise_in_bounds")
            rows_v[r, :] = rows_v[r, :] * splat


def _zero_acc(zero_v, acc, row_base):
    zrow = jnp.zeros((16,), jnp.float32)

    @pl.loop(0, ZR)
    def _(i):
        zero_v[i, :] = zrow

    @pl.loop(0, 8)
    def _(i):
        pltpu.sync_copy(zero_v, acc.at[pl.ds(row_base + i * ZR, ZR)])


@functools.partial(
    pl.kernel,
    out_type=jax.ShapeDtypeStruct((2, NP, 16), jnp.float32),
    mesh=_mesh,
    scratch_types=[
        pltpu.VMEM((SUP1,), jnp.int32),
        pltpu.VMEM((SUP1,), jnp.int32),
        pltpu.VMEM((32, 128), jnp.int32),
        pltpu.VMEM((SUP1,), jnp.float32),
        pltpu.VMEM((128, 16), jnp.float32),
        pltpu.VMEM((ZR, 16), jnp.float32),
        pltpu.VMEM_SHARED((NP, 16), jnp.float32),
        pltpu.SemaphoreType.DMA,
    ],
)
def _phase1(src_hbm, dst_hbm, w_hbm, tab_hbm, out_hbm,
            src_v, src2_v, dst_v, w_v, rows_v, zero_v, acc, sem):
    c = lax.axis_index("c")
    s = lax.axis_index("s")
    row_base = s * RPS
    _zero_acc(zero_v, acc, row_base)
    plsc.subcore_barrier()

    cvec = jnp.full((16,), 0, jnp.int32) + c
    idx_consts = [jnp.full((16,), e, jnp.int32) for e in range(16)]
    ebase = s * (NSUP1 * SUP1)
    cbase = s * (NSUP1 * SUP1 // 128)

    @pl.loop(0, NSUP1)
    def _(sup):
        off = ebase + sup * SUP1
        co = cbase + sup * 32
        pltpu.sync_copy(src_hbm.at[pl.ds(off, SUP1)], src_v)
        pltpu.sync_copy(dst_hbm.at[pl.ds(co, 32)], dst_v)
        pltpu.sync_copy(w_hbm.at[pl.ds(off, SUP1)], w_v)

        @pl.loop(0, SUP1 // 16)
        def _(g):
            sv = src_v[pl.ds(g * 16, 16)]
            src2_v[pl.ds(g * 16, 16)] = sv + sv + cvec

        @pl.loop(0, 32)
        def _(j):
            pltpu.async_copy(
                tab_hbm.at[src2_v.at[pl.ds(j * 128, 128)]], rows_v, sem
            ).wait()
            _scale_chunk(rows_v, w_v, j * 128, idx_consts)
            pltpu.sync_copy(rows_v, acc.at[dst_v.at[j]], add=True)

    plsc.subcore_barrier()
    pltpu.sync_copy(acc.at[pl.ds(row_base, RPS)],
                    out_hbm.at[c, pl.ds(row_base, RPS)])


@functools.partial(
    pl.kernel,
    out_type=jax.ShapeDtypeStruct((2, NP, 16), jnp.float32),
    mesh=_mesh,
    scratch_types=[
        pltpu.VMEM((SUP2,), jnp.int32),
        pltpu.VMEM((16, 128), jnp.int32),
        pltpu.VMEM((SUP2,), jnp.float32),
        pltpu.VMEM((128, 16), jnp.float32),
        pltpu.VMEM((ZR, 16), jnp.float32),
        pltpu.VMEM_SHARED((NP, 16), jnp.float32),
        pltpu.SemaphoreType.DMA,
    ],
)
def _phase2(src_hbm, dst_hbm, w_hbm, tab_hbm, out_hbm,
            src_v, dst_v, w_v, rows_v, zero_v, acc, sem):
    c = lax.axis_index("c")
    s = lax.axis_index("s")
    row_base = s * RPS
    _zero_acc(zero_v, acc, row_base)
    plsc.subcore_barrier()

    idx_consts = [jnp.full((16,), e, jnp.int32) for e in range(16)]
    w_id = c * 16 + s
    ebase = w_id * (NSUP2 * SUP2)
    cbase = w_id * (NSUP2 * SUP2 // 128)

    @pl.loop(0, NSUP2)
    def _(sup):
        off = ebase + sup * SUP2
        co = cbase + sup * 16
        pltpu.sync_copy(src_hbm.at[pl.ds(off, SUP2)], src_v)
        pltpu.sync_copy(dst_hbm.at[pl.ds(co, 16)], dst_v)
        pltpu.sync_copy(w_hbm.at[pl.ds(off, SUP2)], w_v)

        @pl.loop(0, 16)
        def _(j):
            pltpu.async_copy(
                tab_hbm.at[src_v.at[pl.ds(j * 128, 128)]], rows_v, sem
            ).wait()
            _scale_chunk(rows_v, w_v, j * 128, idx_consts)
            pltpu.sync_copy(rows_v, acc.at[dst_v.at[j]], add=True)

    plsc.subcore_barrier()
    pltpu.sync_copy(acc.at[pl.ds(row_base, RPS)],
                    out_hbm.at[c, pl.ds(row_base, RPS)])


def _relu_matmul(h1p, W2):
    blk = NP // 8

    def body(a_ref, b_ref, w_ref, o_ref):
        ha = jnp.maximum(a_ref[0], 0.0)
        hb = jnp.maximum(b_ref[0], 0.0)
        o_ref[...] = (
            jnp.dot(ha, w_ref[0:16, :], preferred_element_type=jnp.float32,
                    precision=lax.Precision.HIGHEST)
            + jnp.dot(hb, w_ref[16:32, :], preferred_element_type=jnp.float32,
                      precision=lax.Precision.HIGHEST)
        )

    return pl.pallas_call(
        body,
        grid=(8,),
        in_specs=[
            pl.BlockSpec((1, blk, 16), lambda i: (0, i, 0)),
            pl.BlockSpec((1, blk, 16), lambda i: (1, i, 0)),
            pl.BlockSpec((32, 16), lambda i: (0, 0)),
        ],
        out_specs=pl.BlockSpec((blk, 16), lambda i: (i, 0)),
        out_shape=jax.ShapeDtypeStruct((NP, 16), jnp.float32),
    )(h1p, h1p, W2)


def _add_parts(parts):
    rows = NP * 16 // 128
    blk = rows // 4
    p = parts.reshape(2, rows, 128)

    def body(p_ref, o_ref):
        o_ref[...] = p_ref[0] + p_ref[1]

    out = pl.pallas_call(
        body,
        grid=(4,),
        in_specs=[pl.BlockSpec((2, blk, 128), lambda i: (0, i, 0))],
        out_specs=pl.BlockSpec((blk, 128), lambda i: (i, 0)),
        out_shape=jax.ShapeDtypeStruct((rows, 128), jnp.float32),
    )(p)
    return out.reshape(NP, 16)


def kernel(edge_index, edge_weight, W1, W2):
    src = edge_index[0]
    dst = edge_index[1]
    pad_idx = (jnp.arange(PAD, dtype=jnp.int32) * 977) % N
    src_p = jnp.concatenate([src, pad_idx])
    dst_p = jnp.concatenate([dst, pad_idx])
    w_p = jnp.concatenate([edge_weight, jnp.zeros((PAD,), jnp.float32)])
    dst2 = dst_p.reshape(EPAD // 128, 128)
    w1r = W1.reshape(2 * N, 16)

    h1p = _phase1(src_p, dst2, w_p, w1r)
    s2 = _relu_matmul(h1p, W2)
    outp = _phase2(src_p, dst2, w_p, s2)
    return _add_parts(outp)[:N]


# speedup vs baseline: 9.8578x; 9.8578x over previous
import functools

import jax
import jax.numpy as jnp
from jax import lax
from jax.experimental import pallas as pl
from jax.experimental.pallas import tpu as pltpu
from jax.experimental.pallas import tpu_sc as plsc

N = 100000
E = 1600000
NP = 100096
RPS = NP // 16
ZR = 782
EPAD = 1638400
PAD = EPAD - E
SUP1 = 4096
NSUP1 = 25
SUP2 = 2048
NSUP2 = 25

_mesh = plsc.VectorSubcoreMesh(core_axis_name="c", subcore_axis_name="s")
_sc_params = pltpu.CompilerParams(use_tc_tiling_on_sc=False)


def _scale_chunk(rows_v, w_v, wbase, idx_consts):
    @pl.loop(0, 8)
    def _(g):
        wv = w_v[pl.ds(wbase + g * 16, 16)]
        for e in range(16):
            r = g * 16 + e
            splat = wv.at[idx_consts[e]].get(mode="promise_in_bounds")
            rows_v[r, :] = rows_v[r, :] * splat


def _zero_acc(zero_v, acc, row_base):
    zrow = jnp.zeros((16,), jnp.float32)

    @pl.loop(0, ZR)
    def _(i):
        zero_v[i, :] = zrow

    @pl.loop(0, 8)
    def _(i):
        pltpu.sync_copy(zero_v, acc.at[pl.ds(row_base + i * ZR, ZR)])


@functools.partial(
    pl.kernel,
    out_type=jax.ShapeDtypeStruct((2, NP, 16), jnp.float32),
    mesh=_mesh,
    compiler_params=_sc_params,
    scratch_types=[
        pltpu.VMEM((SUP1,), jnp.int32),
        pltpu.VMEM((SUP1,), jnp.int32),
        pltpu.VMEM((32, 128), jnp.int32),
        pltpu.VMEM((SUP1,), jnp.float32),
        pltpu.VMEM((128, 16), jnp.float32),
        pltpu.VMEM((ZR, 16), jnp.float32),
        pltpu.VMEM_SHARED((NP, 16), jnp.float32),
        pltpu.SemaphoreType.DMA,
    ],
)
def _phase1(src_hbm, dst_hbm, w_hbm, tab_hbm, out_hbm,
            src_v, src2_v, dst_v, w_v, rows_v, zero_v, acc, sem):
    c = lax.axis_index("c")
    s = lax.axis_index("s")
    row_base = s * RPS
    _zero_acc(zero_v, acc, row_base)
    plsc.subcore_barrier()

    cvec = jnp.full((16,), 0, jnp.int32) + c
    idx_consts = [jnp.full((16,), e, jnp.int32) for e in range(16)]
    ebase = s * (NSUP1 * SUP1)
    cbase = s * (NSUP1 * SUP1 // 128)

    @pl.loop(0, NSUP1)
    def _(sup):
        off = ebase + sup * SUP1
        co = cbase + sup * 32
        pltpu.sync_copy(src_hbm.at[pl.ds(off, SUP1)], src_v)
        pltpu.sync_copy(dst_hbm.at[pl.ds(co, 32)], dst_v)
        pltpu.sync_copy(w_hbm.at[pl.ds(off, SUP1)], w_v)

        @pl.loop(0, SUP1 // 16)
        def _(g):
            sv = src_v[pl.ds(g * 16, 16)]
            src2_v[pl.ds(g * 16, 16)] = sv + sv + cvec

        @pl.loop(0, 32)
        def _(j):
            pltpu.async_copy(
                tab_hbm.at[src2_v.at[pl.ds(j * 128, 128)]], rows_v, sem
            ).wait()
            _scale_chunk(rows_v, w_v, j * 128, idx_consts)
            pltpu.sync_copy(rows_v, acc.at[dst_v.at[j]], add=True)

    plsc.subcore_barrier()
    pltpu.sync_copy(acc.at[pl.ds(row_base, RPS)],
                    out_hbm.at[c, pl.ds(row_base, RPS)])


@functools.partial(
    pl.kernel,
    out_type=jax.ShapeDtypeStruct((2, NP, 16), jnp.float32),
    mesh=_mesh,
    compiler_params=_sc_params,
    scratch_types=[
        pltpu.VMEM((SUP2,), jnp.int32),
        pltpu.VMEM((16, 128), jnp.int32),
        pltpu.VMEM((SUP2,), jnp.float32),
        pltpu.VMEM((128, 16), jnp.float32),
        pltpu.VMEM((ZR, 16), jnp.float32),
        pltpu.VMEM_SHARED((NP, 16), jnp.float32),
        pltpu.SemaphoreType.DMA,
    ],
)
def _phase2(src_hbm, dst_hbm, w_hbm, tab_hbm, out_hbm,
            src_v, dst_v, w_v, rows_v, zero_v, acc, sem):
    c = lax.axis_index("c")
    s = lax.axis_index("s")
    row_base = s * RPS
    _zero_acc(zero_v, acc, row_base)
    plsc.subcore_barrier()

    idx_consts = [jnp.full((16,), e, jnp.int32) for e in range(16)]
    w_id = c * 16 + s
    ebase = w_id * (NSUP2 * SUP2)
    cbase = w_id * (NSUP2 * SUP2 // 128)

    @pl.loop(0, NSUP2)
    def _(sup):
        off = ebase + sup * SUP2
        co = cbase + sup * 16
        pltpu.sync_copy(src_hbm.at[pl.ds(off, SUP2)], src_v)
        pltpu.sync_copy(dst_hbm.at[pl.ds(co, 16)], dst_v)
        pltpu.sync_copy(w_hbm.at[pl.ds(off, SUP2)], w_v)

        @pl.loop(0, 16)
        def _(j):
            pltpu.async_copy(
                tab_hbm.at[src_v.at[pl.ds(j * 128, 128)]], rows_v, sem
            ).wait()
            _scale_chunk(rows_v, w_v, j * 128, idx_consts)
            pltpu.sync_copy(rows_v, acc.at[dst_v.at[j]], add=True)

    plsc.subcore_barrier()
    pltpu.sync_copy(acc.at[pl.ds(row_base, RPS)],
                    out_hbm.at[c, pl.ds(row_base, RPS)])


def _relu_matmul(h1p, W2):
    blk = NP // 32

    def body(a_ref, b_ref, w_ref, o_ref):
        ha = jnp.maximum(a_ref[0], 0.0)
        hb = jnp.maximum(b_ref[0], 0.0)
        o_ref[...] = (
            jnp.dot(ha, w_ref[0:16, :], preferred_element_type=jnp.float32,
                    precision=lax.Precision.HIGHEST)
            + jnp.dot(hb, w_ref[16:32, :], preferred_element_type=jnp.float32,
                      precision=lax.Precision.HIGHEST)
        )

    return pl.pallas_call(
        body,
        grid=(32,),
        in_specs=[
            pl.BlockSpec((1, blk, 16), lambda i: (0, i, 0)),
            pl.BlockSpec((1, blk, 16), lambda i: (1, i, 0)),
            pl.BlockSpec((32, 16), lambda i: (0, 0)),
        ],
        out_specs=pl.BlockSpec((blk, 16), lambda i: (i, 0)),
        out_shape=jax.ShapeDtypeStruct((NP, 16), jnp.float32),
    )(h1p, h1p, W2)


def _add_parts(parts):
    rows = NP * 16 // 128
    blk = rows // 4
    p = parts.reshape(2, rows, 128)

    def body(p_ref, o_ref):
        o_ref[...] = p_ref[0] + p_ref[1]

    out = pl.pallas_call(
        body,
        grid=(4,),
        in_specs=[pl.BlockSpec((2, blk, 128), lambda i: (0, i, 0))],
        out_specs=pl.BlockSpec((blk, 128), lambda i: (i, 0)),
        out_shape=jax.ShapeDtypeStruct((rows, 128), jnp.float32),
    )(p)
    return out.reshape(NP, 16)


def kernel(edge_index, edge_weight, W1, W2):
    src = edge_index[0]
    dst = edge_index[1]
    pad_idx = (jnp.arange(PAD, dtype=jnp.int32) * 977) % N
    src_p = jnp.concatenate([src, pad_idx])
    dst_p = jnp.concatenate([dst, pad_idx])
    w_p = jnp.concatenate([edge_weight, jnp.zeros((PAD,), jnp.float32)])
    dst2 = dst_p.reshape(EPAD // 128, 128)
    w1r = W1.reshape(2 * N, 16)

    h1p = _phase1(src_p, dst2, w_p, w1r)
    s2 = _relu_matmul(h1p, W2)
    outp = _phase2(src_p, dst2, w_p, s2)
    return _add_parts(outp)[:N]


# speedup vs baseline: 17.8835x; 1.8141x over previous
import functools

import jax
import jax.numpy as jnp
from jax import lax
from jax.experimental import pallas as pl
from jax.experimental.pallas import tpu as pltpu
from jax.experimental.pallas import tpu_sc as plsc

N = 100000
E = 1600000
NP = 100096
RPS = NP // 16
ZR = 782
EPAD = 1638400
PAD = EPAD - E

_mesh = plsc.VectorSubcoreMesh(core_axis_name="c", subcore_axis_name="s")
_sc_params = pltpu.CompilerParams(use_tc_tiling_on_sc=False)


def _make_phase(sup, nsup, bat, transform):
    nch = sup // 128
    npair = nch // bat // 2

    @functools.partial(
        pl.kernel,
        out_type=jax.ShapeDtypeStruct((2, NP, 16), jnp.float32),
        mesh=_mesh,
        compiler_params=_sc_params,
        scratch_types=[
            pltpu.VMEM((sup,), jnp.int32),
            pltpu.VMEM((nch, 128), jnp.int32),
            pltpu.VMEM((sup,), jnp.float32),
            pltpu.VMEM((bat * 128, 16), jnp.float32),
            pltpu.VMEM((bat * 128, 16), jnp.float32),
            pltpu.VMEM_SHARED((NP, 16), jnp.float32),
            pltpu.SemaphoreType.DMA,
            pltpu.SemaphoreType.DMA,
        ],
    )
    def phase(src_hbm, dst_hbm, w_hbm, tab_hbm, out_hbm,
              src_v, dst_v, w_v, bufA, bufB, acc, semA, semB):
        c = lax.axis_index("c")
        s = lax.axis_index("s")
        row_base = s * RPS
        zrow = jnp.zeros((16,), jnp.float32)

        @pl.loop(0, 391)
        def _(i):
            bufA[i, :] = zrow

        @pl.loop(0, 16)
        def _(i):
            pltpu.sync_copy(bufA.at[pl.ds(0, 391)],
                            acc.at[pl.ds(row_base + i * 391, 391)])

        plsc.subcore_barrier()

        idx_consts = [jnp.full((16,), e, jnp.int32) for e in range(16)]
        cvec = jnp.full((16,), 0, jnp.int32) + c
        w_id = s if transform else c * 16 + s
        wbase = w_id * (nsup * sup)

        def copy(boff, k, buf, sem):
            return pltpu.make_async_copy(
                tab_hbm.at[src_v.at[pl.ds((boff + k) * 128, 128)]],
                buf.at[pl.ds(k * 128, 128)], sem)

        def fire(boff, buf, sem):
            for k in range(bat):
                copy(boff, k, buf, sem).start()

        def drain_process(boff, buf, sem):
            for k in range(bat):
                copy(boff, k, buf, sem).wait()
            for k in range(bat):
                @pl.loop(0, 8)
                def _(g):
                    wv = w_v[pl.ds((boff + k) * 128 + g * 16, 16)]
                    for e in range(16):
                        r = k * 128 + g * 16 + e
                        splat = wv.at[idx_consts[e]].get(
                            mode="promise_in_bounds")
                        buf[r, :] = buf[r, :] * splat
                pltpu.sync_copy(buf.at[pl.ds(k * 128, 128)],
                                acc.at[dst_v.at[boff + k]], add=True)

        @pl.loop(0, nsup)
        def _(sup_i):
            off = wbase + sup_i * sup
            co = off // 128
            pltpu.sync_copy(src_hbm.at[pl.ds(off, sup)], src_v)
            pltpu.sync_copy(dst_hbm.at[pl.ds(co, nch)], dst_v)
            pltpu.sync_copy(w_hbm.at[pl.ds(off, sup)], w_v)

            if transform:
                @pl.loop(0, sup // 16)
                def _(g):
                    sv = src_v[pl.ds(g * 16, 16)]
                    src_v[pl.ds(g * 16, 16)] = sv + sv + cvec

            fire(0, bufA, semA)

            @pl.loop(0, npair)
            def _(p):
                b0 = p * (2 * bat)
                fire(b0 + bat, bufB, semB)
                drain_process(b0, bufA, semA)

                @pl.when(p < npair - 1)
                def _():
                    fire(b0 + 2 * bat, bufA, semA)

                drain_process(b0 + bat, bufB, semB)

        plsc.subcore_barrier()
        pltpu.sync_copy(acc.at[pl.ds(row_base, RPS)],
                        out_hbm.at[c, pl.ds(row_base, RPS)])

    return phase


_phase1 = _make_phase(sup=4096, nsup=25, bat=4, transform=True)
_phase2 = _make_phase(sup=2560, nsup=20, bat=5, transform=False)


def _relu_matmul(h1p, W2):
    blk = NP // 32

    def body(a_ref, b_ref, w_ref, o_ref):
        ha = jnp.maximum(a_ref[0], 0.0)
        hb = jnp.maximum(b_ref[0], 0.0)
        o_ref[...] = (
            jnp.dot(ha, w_ref[0:16, :], preferred_element_type=jnp.float32,
                    precision=lax.Precision.HIGHEST)
            + jnp.dot(hb, w_ref[16:32, :], preferred_element_type=jnp.float32,
                      precision=lax.Precision.HIGHEST)
        )

    return pl.pallas_call(
        body,
        grid=(32,),
        in_specs=[
            pl.BlockSpec((1, blk, 16), lambda i: (0, i, 0)),
            pl.BlockSpec((1, blk, 16), lambda i: (1, i, 0)),
            pl.BlockSpec((32, 16), lambda i: (0, 0)),
        ],
        out_specs=pl.BlockSpec((blk, 16), lambda i: (i, 0)),
        out_shape=jax.ShapeDtypeStruct((NP, 16), jnp.float32),
    )(h1p, h1p, W2)


def _add_parts(parts):
    rows = NP * 16 // 128
    blk = rows // 4
    p = parts.reshape(2, rows, 128)

    def body(p_ref, o_ref):
        o_ref[...] = p_ref[0] + p_ref[1]

    out = pl.pallas_call(
        body,
        grid=(4,),
        in_specs=[pl.BlockSpec((2, blk, 128), lambda i: (0, i, 0))],
        out_specs=pl.BlockSpec((blk, 128), lambda i: (i, 0)),
        out_shape=jax.ShapeDtypeStruct((rows, 128), jnp.float32),
    )(p)
    return out.reshape(NP, 16)


def kernel(edge_index, edge_weight, W1, W2):
    src = edge_index[0]
    dst = edge_index[1]
    pad_idx = (jnp.arange(PAD, dtype=jnp.int32) * 977) % N
    src_p = jnp.concatenate([src, pad_idx])
    dst_p = jnp.concatenate([dst, pad_idx])
    w_p = jnp.concatenate([edge_weight, jnp.zeros((PAD,), jnp.float32)])
    dst2 = dst_p.reshape(EPAD // 128, 128)
    w1r = W1.reshape(2 * N, 16)

    h1p = _phase1(src_p, dst2, w_p, w1r)
    s2 = _relu_matmul(h1p, W2)
    outp = _phase2(src_p, dst2, w_p, s2)
    return _add_parts(outp)[:N]


# speedup vs baseline: 18.7780x; 1.0500x over previous
import functools

import jax
import jax.numpy as jnp
from jax import lax
from jax.experimental import pallas as pl
from jax.experimental.pallas import tpu as pltpu
from jax.experimental.pallas import tpu_sc as plsc

N = 100000
E = 1600000
NP = 100096
RPS = NP // 16
ZR = 782
EPAD = 1638400
PAD = EPAD - E

_mesh = plsc.VectorSubcoreMesh(core_axis_name="c", subcore_axis_name="s")
_sc_params = pltpu.CompilerParams(use_tc_tiling_on_sc=False)


def _make_phase(sup, nsup, bat, transform):
    nch = sup // 128
    npair = nch // bat // 2

    @functools.partial(
        pl.kernel,
        out_type=jax.ShapeDtypeStruct((2, NP, 16), jnp.float32),
        mesh=_mesh,
        compiler_params=_sc_params,
        scratch_types=[
            pltpu.VMEM((sup,), jnp.int32),
            pltpu.VMEM((nch, 128), jnp.int32),
            pltpu.VMEM((sup,), jnp.float32),
            pltpu.VMEM((bat * 128, 16), jnp.float32),
            pltpu.VMEM((bat * 128, 16), jnp.float32),
            pltpu.VMEM_SHARED((NP, 16), jnp.float32),
            pltpu.SemaphoreType.DMA,
            pltpu.SemaphoreType.DMA,
        ],
    )
    def phase(src_hbm, dst_hbm, w_hbm, tab_hbm, out_hbm,
              src_v, dst_v, w_v, bufA, bufB, acc, semA, semB):
        c = lax.axis_index("c")
        s = lax.axis_index("s")
        row_base = s * RPS
        zrow = jnp.zeros((16,), jnp.float32)

        @plsc.parallel_loop(0, 391, unroll=4)
        def _(i):
            bufA[i, :] = zrow

        @pl.loop(0, 16)
        def _(i):
            pltpu.sync_copy(bufA.at[pl.ds(0, 391)],
                            acc.at[pl.ds(row_base + i * 391, 391)])

        plsc.subcore_barrier()

        idx_consts = [jnp.full((16,), e, jnp.int32) for e in range(16)]
        cvec = jnp.full((16,), 0, jnp.int32) + c
        w_id = s if transform else c * 16 + s
        wbase = w_id * (nsup * sup)

        def copy(boff, k, buf, sem):
            return pltpu.make_async_copy(
                tab_hbm.at[src_v.at[pl.ds((boff + k) * 128, 128)]],
                buf.at[pl.ds(k * 128, 128)], sem)

        def fire(boff, buf, sem):
            for k in range(bat):
                copy(boff, k, buf, sem).start()

        def drain_process(boff, buf, sem):
            for k in range(bat):
                copy(boff, k, buf, sem).wait()
            for k in range(bat):
                @plsc.parallel_loop(0, 8, unroll=2)
                def _(g):
                    wv = w_v[pl.ds((boff + k) * 128 + g * 16, 16)]
                    for e in range(16):
                        r = k * 128 + g * 16 + e
                        splat = wv.at[idx_consts[e]].get(
                            mode="promise_in_bounds")
                        buf[r, :] = buf[r, :] * splat
                pltpu.sync_copy(buf.at[pl.ds(k * 128, 128)],
                                acc.at[dst_v.at[boff + k]], add=True)

        @pl.loop(0, nsup)
        def _(sup_i):
            off = wbase + sup_i * sup
            co = off // 128
            pltpu.sync_copy(src_hbm.at[pl.ds(off, sup)], src_v)
            pltpu.sync_copy(dst_hbm.at[pl.ds(co, nch)], dst_v)
            pltpu.sync_copy(w_hbm.at[pl.ds(off, sup)], w_v)

            if transform:
                @plsc.parallel_loop(0, sup // 16, unroll=4)
                def _(g):
                    sv = src_v[pl.ds(g * 16, 16)]
                    src_v[pl.ds(g * 16, 16)] = sv + sv + cvec

            fire(0, bufA, semA)

            @pl.loop(0, npair)
            def _(p):
                b0 = p * (2 * bat)
                fire(b0 + bat, bufB, semB)
                drain_process(b0, bufA, semA)

                @pl.when(p < npair - 1)
                def _():
                    fire(b0 + 2 * bat, bufA, semA)

                drain_process(b0 + bat, bufB, semB)

        plsc.subcore_barrier()
        pltpu.sync_copy(acc.at[pl.ds(row_base, RPS)],
                        out_hbm.at[c, pl.ds(row_base, RPS)])

    return phase


_phase1 = _make_phase(sup=4096, nsup=25, bat=4, transform=True)
_phase2 = _make_phase(sup=2560, nsup=20, bat=5, transform=False)


def _relu_matmul(h1p, W2):
    blk = NP // 32

    def body(a_ref, b_ref, w_ref, o_ref):
        ha = jnp.maximum(a_ref[0], 0.0)
        hb = jnp.maximum(b_ref[0], 0.0)
        o_ref[...] = (
            jnp.dot(ha, w_ref[0:16, :], preferred_element_type=jnp.float32,
                    precision=lax.Precision.HIGHEST)
            + jnp.dot(hb, w_ref[16:32, :], preferred_element_type=jnp.float32,
                      precision=lax.Precision.HIGHEST)
        )

    return pl.pallas_call(
        body,
        grid=(32,),
        in_specs=[
            pl.BlockSpec((1, blk, 16), lambda i: (0, i, 0)),
            pl.BlockSpec((1, blk, 16), lambda i: (1, i, 0)),
            pl.BlockSpec((32, 16), lambda i: (0, 0)),
        ],
        out_specs=pl.BlockSpec((blk, 16), lambda i: (i, 0)),
        out_shape=jax.ShapeDtypeStruct((NP, 16), jnp.float32),
    )(h1p, h1p, W2)


def _add_parts(parts):
    rows = NP * 16 // 128
    blk = rows // 4
    p = parts.reshape(2, rows, 128)

    def body(p_ref, o_ref):
        o_ref[...] = p_ref[0] + p_ref[1]

    out = pl.pallas_call(
        body,
        grid=(4,),
        in_specs=[pl.BlockSpec((2, blk, 128), lambda i: (0, i, 0))],
        out_specs=pl.BlockSpec((blk, 128), lambda i: (i, 0)),
        out_shape=jax.ShapeDtypeStruct((rows, 128), jnp.float32),
    )(p)
    return out.reshape(NP, 16)


def kernel(edge_index, edge_weight, W1, W2):
    src = edge_index[0]
    dst = edge_index[1]
    pad_idx = (jnp.arange(PAD, dtype=jnp.int32) * 977) % N
    src_p = jnp.concatenate([src, pad_idx])
    dst_p = jnp.concatenate([dst, pad_idx])
    w_p = jnp.concatenate([edge_weight, jnp.zeros((PAD,), jnp.float32)])
    dst2 = dst_p.reshape(EPAD // 128, 128)
    w1r = W1.reshape(2 * N, 16)

    h1p = _phase1(src_p, dst2, w_p, w1r)
    s2 = _relu_matmul(h1p, W2)
    outp = _phase2(src_p, dst2, w_p, s2)
    return _add_parts(outp)[:N]


# speedup vs baseline: 24.9151x; 1.3268x over previous
import functools

import jax
import jax.numpy as jnp
from jax import lax
from jax.experimental import pallas as pl
from jax.experimental.pallas import tpu as pltpu
from jax.experimental.pallas import tpu_sc as plsc

N = 100000
E = 1600000
NP = 100096
RPS = NP // 16
ZR = 782
EPAD = 1638400
PAD = EPAD - E

_mesh = plsc.VectorSubcoreMesh(core_axis_name="c", subcore_axis_name="s")
_sc_params = pltpu.CompilerParams(use_tc_tiling_on_sc=False)


def _make_phase(sup, nsup, bat, transform):
    nch = sup // 128
    npair = nch // bat // 2

    @functools.partial(
        pl.kernel,
        out_type=jax.ShapeDtypeStruct((2, NP, 16), jnp.float32),
        mesh=_mesh,
        compiler_params=_sc_params,
        scratch_types=[
            pltpu.VMEM((sup,), jnp.int32),
            pltpu.VMEM((nch, 128), jnp.int32),
            pltpu.VMEM((sup,), jnp.float32),
            pltpu.VMEM((bat * 128, 16), jnp.float32),
            pltpu.VMEM((bat * 128, 16), jnp.float32),
            pltpu.VMEM_SHARED((NP, 16), jnp.float32),
            pltpu.SemaphoreType.DMA,
            pltpu.SemaphoreType.DMA,
        ],
    )
    def phase(src_hbm, dst_hbm, w_hbm, tab_hbm, out_hbm,
              src_v, dst_v, w_v, bufA, bufB, acc, semA, semB):
        c = lax.axis_index("c")
        s = lax.axis_index("s")
        row_base = s * RPS
        zrow = jnp.zeros((16,), jnp.float32)

        @plsc.parallel_loop(0, 391, unroll=4)
        def _(i):
            bufA[i, :] = zrow

        @pl.loop(0, 16)
        def _(i):
            pltpu.sync_copy(bufA.at[pl.ds(0, 391)],
                            acc.at[pl.ds(row_base + i * 391, 391)])

        plsc.subcore_barrier()

        idx_consts = [jnp.full((16,), e, jnp.int32) for e in range(16)]
        cvec = jnp.full((16,), 0, jnp.int32) + c
        w_id = s if transform else c * 16 + s
        wbase = w_id * (nsup * sup)

        def copy(boff, k, buf, sem):
            return pltpu.make_async_copy(
                tab_hbm.at[src_v.at[pl.ds((boff + k) * 128, 128)]],
                buf.at[pl.ds(k * 128, 128)], sem)

        def fire(boff, buf, sem):
            for k in range(bat):
                copy(boff, k, buf, sem).start()

        def drain_process(boff, buf, sem):
            for k in range(bat):
                copy(boff, k, buf, sem).wait()
            for k in range(bat):
                @plsc.parallel_loop(0, 8, unroll=2)
                def _(g):
                    wv = w_v[pl.ds((boff + k) * 128 + g * 16, 16)]
                    for e in range(16):
                        r = k * 128 + g * 16 + e
                        splat = wv.at[idx_consts[e]].get(
                            mode="promise_in_bounds")
                        buf[r, :] = buf[r, :] * splat
                pltpu.sync_copy(buf.at[pl.ds(k * 128, 128)],
                                acc.at[dst_v.at[boff + k]], add=True)

        @pl.loop(0, nsup)
        def _(sup_i):
            off = wbase + sup_i * sup
            co = off // 128
            pltpu.sync_copy(src_hbm.at[pl.ds(off, sup)], src_v)
            pltpu.sync_copy(dst_hbm.at[pl.ds(co, nch)], dst_v)
            pltpu.sync_copy(w_hbm.at[pl.ds(off, sup)], w_v)

            if transform:
                @plsc.parallel_loop(0, sup // 16, unroll=4)
                def _(g):
                    sv = src_v[pl.ds(g * 16, 16)]
                    src_v[pl.ds(g * 16, 16)] = sv + sv + cvec

            fire(0, bufA, semA)

            @pl.loop(0, npair)
            def _(p):
                b0 = p * (2 * bat)
                fire(b0 + bat, bufB, semB)
                drain_process(b0, bufA, semA)

                @pl.when(p < npair - 1)
                def _():
                    fire(b0 + 2 * bat, bufA, semA)

                drain_process(b0 + bat, bufB, semB)

        plsc.subcore_barrier()
        pltpu.sync_copy(acc.at[pl.ds(row_base, RPS)],
                        out_hbm.at[c, pl.ds(row_base, RPS)])

    return phase


_phase1 = _make_phase(sup=4096, nsup=25, bat=4, transform=True)
_phase2 = _make_phase(sup=2560, nsup=20, bat=5, transform=False)


def _relu_matmul(h1p, W2):
    rows = NP * 16 // 128
    blk = rows // 4
    p = h1p.reshape(2, rows, 128)
    eye8 = jnp.eye(8, dtype=jnp.float32)
    bd_a = jnp.kron(eye8, W2[:16, :])
    bd_b = jnp.kron(eye8, W2[16:, :])

    def body(p_ref, wa_ref, wb_ref, o_ref):
        ha = jnp.maximum(p_ref[0], 0.0)
        hb = jnp.maximum(p_ref[1], 0.0)
        o_ref[...] = (
            jnp.dot(ha, wa_ref[...], preferred_element_type=jnp.float32,
                    precision=lax.Precision.HIGHEST)
            + jnp.dot(hb, wb_ref[...], preferred_element_type=jnp.float32,
                      precision=lax.Precision.HIGHEST)
        )

    out = pl.pallas_call(
        body,
        grid=(4,),
        in_specs=[
            pl.BlockSpec((2, blk, 128), lambda i: (0, i, 0)),
            pl.BlockSpec((128, 128), lambda i: (0, 0)),
            pl.BlockSpec((128, 128), lambda i: (0, 0)),
        ],
        out_specs=pl.BlockSpec((blk, 128), lambda i: (i, 0)),
        out_shape=jax.ShapeDtypeStruct((rows, 128), jnp.float32),
    )(p, bd_a, bd_b)
    return out.reshape(NP, 16)


def _add_parts(parts):
    rows = NP * 16 // 128
    blk = rows // 4
    p = parts.reshape(2, rows, 128)

    def body(p_ref, o_ref):
        o_ref[...] = p_ref[0] + p_ref[1]

    out = pl.pallas_call(
        body,
        grid=(4,),
        in_specs=[pl.BlockSpec((2, blk, 128), lambda i: (0, i, 0))],
        out_specs=pl.BlockSpec((blk, 128), lambda i: (i, 0)),
        out_shape=jax.ShapeDtypeStruct((rows, 128), jnp.float32),
    )(p)
    return out.reshape(NP, 16)


def kernel(edge_index, edge_weight, W1, W2):
    src = edge_index[0]
    dst = edge_index[1]
    pad_idx = (jnp.arange(PAD, dtype=jnp.int32) * 977) % N
    src_p = jnp.concatenate([src, pad_idx])
    dst_p = jnp.concatenate([dst, pad_idx])
    w_p = jnp.concatenate([edge_weight, jnp.zeros((PAD,), jnp.float32)])
    dst2 = dst_p.reshape(EPAD // 128, 128)
    w1r = W1.reshape(2 * N, 16)

    h1p = _phase1(src_p, dst2, w_p, w1r)
    s2 = _relu_matmul(h1p, W2)
    outp = _phase2(src_p, dst2, w_p, s2)
    return _add_parts(outp)[:N]


# speedup vs baseline: 27.0017x; 1.0837x over previous
import functools

import jax
import jax.numpy as jnp
from jax import lax
from jax.experimental import pallas as pl
from jax.experimental.pallas import tpu as pltpu
from jax.experimental.pallas import tpu_sc as plsc

N = 100000
E = 1600000
NP = 100096
RPS = NP // 16
ZR = 782
EPAD = 1638400
PAD = EPAD - E

_mesh = plsc.VectorSubcoreMesh(core_axis_name="c", subcore_axis_name="s")
_sc_params = pltpu.CompilerParams(use_tc_tiling_on_sc=False)


def _make_phase(sup, nsup, bat, transform):
    nch = sup // 128
    npair = nch // bat // 2

    @functools.partial(
        pl.kernel,
        out_type=jax.ShapeDtypeStruct((2, NP, 16), jnp.float32),
        mesh=_mesh,
        compiler_params=_sc_params,
        scratch_types=[
            pltpu.VMEM((sup,), jnp.int32),
            pltpu.VMEM((nch, 128), jnp.int32),
            pltpu.VMEM((sup,), jnp.float32),
            pltpu.VMEM((bat * 128, 16), jnp.float32),
            pltpu.VMEM((bat * 128, 16), jnp.float32),
            pltpu.VMEM_SHARED((NP, 16), jnp.float32),
            pltpu.SemaphoreType.DMA,
            pltpu.SemaphoreType.DMA,
        ],
    )
    def phase(*refs):
        if transform:
            (srcA_hbm, srcB_hbm, dst_hbm, w_hbm, tab_hbm, out_hbm,
             src_v, dst_v, w_v, bufA, bufB, acc, semA, semB) = refs
        else:
            (srcA_hbm, dst_hbm, w_hbm, tab_hbm, out_hbm,
             src_v, dst_v, w_v, bufA, bufB, acc, semA, semB) = refs
            srcB_hbm = srcA_hbm
        c = lax.axis_index("c")
        s = lax.axis_index("s")
        row_base = s * RPS
        zrow = jnp.zeros((16,), jnp.float32)

        @plsc.parallel_loop(0, 391, unroll=4)
        def _(i):
            bufA[i, :] = zrow

        @pl.loop(0, 16)
        def _(i):
            pltpu.sync_copy(bufA.at[pl.ds(0, 391)],
                            acc.at[pl.ds(row_base + i * 391, 391)])

        plsc.subcore_barrier()

        idx_consts = [jnp.full((16,), e, jnp.int32) for e in range(16)]
        w_id = s if transform else c * 16 + s
        wbase = w_id * (nsup * sup)

        def copy(boff, k, buf, sem):
            return pltpu.make_async_copy(
                tab_hbm.at[src_v.at[pl.ds((boff + k) * 128, 128)]],
                buf.at[pl.ds(k * 128, 128)], sem)

        def fire(boff, buf, sem):
            for k in range(bat):
                copy(boff, k, buf, sem).start()

        def drain_process(boff, buf, sem):
            for k in range(bat):
                copy(boff, k, buf, sem).wait()
            for k in range(bat):
                @plsc.parallel_loop(0, 8, unroll=2)
                def _(g):
                    wv = w_v[pl.ds((boff + k) * 128 + g * 16, 16)]
                    for e in range(16):
                        r = k * 128 + g * 16 + e
                        splat = wv.at[idx_consts[e]].get(
                            mode="promise_in_bounds")
                        buf[r, :] = buf[r, :] * splat
                pltpu.sync_copy(buf.at[pl.ds(k * 128, 128)],
                                acc.at[dst_v.at[boff + k]], add=True)

        @pl.loop(0, nsup)
        def _(sup_i):
            off = wbase + sup_i * sup
            co = off // 128
            if transform:
                @pl.when(c == 0)
                def _():
                    pltpu.sync_copy(srcA_hbm.at[pl.ds(off, sup)], src_v)

                @pl.when(c == 1)
                def _():
                    pltpu.sync_copy(srcB_hbm.at[pl.ds(off, sup)], src_v)
            else:
                pltpu.sync_copy(srcA_hbm.at[pl.ds(off, sup)], src_v)
            pltpu.sync_copy(dst_hbm.at[pl.ds(co, nch)], dst_v)
            pltpu.sync_copy(w_hbm.at[pl.ds(off, sup)], w_v)

            fire(0, bufA, semA)

            @pl.loop(0, npair)
            def _(p):
                b0 = p * (2 * bat)
                fire(b0 + bat, bufB, semB)
                drain_process(b0, bufA, semA)

                @pl.when(p < npair - 1)
                def _():
                    fire(b0 + 2 * bat, bufA, semA)

                drain_process(b0 + bat, bufB, semB)

        plsc.subcore_barrier()
        pltpu.sync_copy(acc.at[pl.ds(row_base, RPS)],
                        out_hbm.at[c, pl.ds(row_base, RPS)])

    return phase


_phase1 = _make_phase(sup=4096, nsup=25, bat=4, transform=True)
_phase2 = _make_phase(sup=2560, nsup=20, bat=5, transform=False)


def _relu_matmul(h1p, W2):
    rows = NP * 16 // 128
    blk = rows // 4
    p = h1p.reshape(2, rows, 128)
    eye8 = jnp.eye(8, dtype=jnp.float32)
    bd_a = jnp.kron(eye8, W2[:16, :])
    bd_b = jnp.kron(eye8, W2[16:, :])

    def body(p_ref, wa_ref, wb_ref, o_ref):
        ha = jnp.maximum(p_ref[0], 0.0)
        hb = jnp.maximum(p_ref[1], 0.0)
        o_ref[...] = (
            jnp.dot(ha, wa_ref[...], preferred_element_type=jnp.float32,
                    precision=lax.Precision.HIGHEST)
            + jnp.dot(hb, wb_ref[...], preferred_element_type=jnp.float32,
                      precision=lax.Precision.HIGHEST)
        )

    out = pl.pallas_call(
        body,
        grid=(4,),
        in_specs=[
            pl.BlockSpec((2, blk, 128), lambda i: (0, i, 0)),
            pl.BlockSpec((128, 128), lambda i: (0, 0)),
            pl.BlockSpec((128, 128), lambda i: (0, 0)),
        ],
        out_specs=pl.BlockSpec((blk, 128), lambda i: (i, 0)),
        out_shape=jax.ShapeDtypeStruct((rows, 128), jnp.float32),
    )(p, bd_a, bd_b)
    return out.reshape(NP, 16)


def _add_parts(parts):
    rows = NP * 16 // 128
    blk = rows // 4
    p = parts.reshape(2, rows, 128)

    def body(p_ref, o_ref):
        o_ref[...] = p_ref[0] + p_ref[1]

    out = pl.pallas_call(
        body,
        grid=(4,),
        in_specs=[pl.BlockSpec((2, blk, 128), lambda i: (0, i, 0))],
        out_specs=pl.BlockSpec((blk, 128), lambda i: (i, 0)),
        out_shape=jax.ShapeDtypeStruct((rows, 128), jnp.float32),
    )(p)
    return out.reshape(NP, 16)


def _edge_prep(edge_index, edge_weight):
    rb = 1600
    eb = rb * 128
    nr = EPAD // 128

    def body(ei_ref, w_ref, o2a, o2b, osr, odr, owr):
        i = pl.program_id(0)
        s = ei_ref[0, :].reshape(rb, 128)
        d = ei_ref[1, :].reshape(rb, 128)
        w = w_ref[...]
        io = (jax.lax.broadcasted_iota(jnp.int32, (rb, 128), 0) * 128
              + jax.lax.broadcasted_iota(jnp.int32, (rb, 128), 1) + i * eb)
        padv = (io * 977) % N
        valid = io < E
        sv = jnp.where(valid, s, padv)
        dv = jnp.where(valid, d, padv)
        wv = jnp.where(valid, w, jnp.float32(0.0))
        o2a[...] = sv + sv
        o2b[...] = sv + sv + 1
        osr[...] = sv
        odr[...] = dv
        owr[...] = wv

    i32out = jax.ShapeDtypeStruct((nr, 128), jnp.int32)
    return pl.pallas_call(
        body,
        grid=(EPAD // eb,),
        in_specs=[
            pl.BlockSpec((2, eb), lambda i: (0, i)),
            pl.BlockSpec((rb, 128), lambda i: (i, 0)),
        ],
        out_specs=[pl.BlockSpec((rb, 128), lambda i: (i, 0))] * 5,
        out_shape=[i32out, i32out, i32out, i32out,
                   jax.ShapeDtypeStruct((nr, 128), jnp.float32)],
    )(edge_index, edge_weight.reshape(E // 128, 128))


def kernel(edge_index, edge_weight, W1, W2):
    s2a, s2b, osr, odr, owr = _edge_prep(edge_index, edge_weight)
    s2a = s2a.reshape(EPAD)
    s2b = s2b.reshape(EPAD)
    osr = osr.reshape(EPAD)
    owr = owr.reshape(EPAD)
    w1r = W1.reshape(2 * N, 16)

    h1p = _phase1(s2a, s2b, odr, owr, w1r)
    s2 = _relu_matmul(h1p, W2)
    outp = _phase2(osr, odr, owr, s2)
    return _add_parts(outp)[:N]


# speedup vs baseline: 30.5318x; 1.1307x over previous
import functools

import jax
import jax.numpy as jnp
from jax import lax
from jax.experimental import pallas as pl
from jax.experimental.pallas import tpu as pltpu
from jax.experimental.pallas import tpu_sc as plsc

N = 100000
E = 1600000
NP = 100096
RPS = NP // 16
ZR = 782
EPAD = 1638400
PAD = EPAD - E

_mesh = plsc.VectorSubcoreMesh(core_axis_name="c", subcore_axis_name="s")
_sc_params = pltpu.CompilerParams(use_tc_tiling_on_sc=False)


def _make_phase(sup, nsup, bat, transform):
    nch = sup // 128
    npair = nch // bat // 2

    @functools.partial(
        pl.kernel,
        out_type=jax.ShapeDtypeStruct((2, NP, 16), jnp.float32),
        mesh=_mesh,
        compiler_params=_sc_params,
        scratch_types=[
            pltpu.VMEM((sup,), jnp.int32),
            pltpu.VMEM((nch, 128), jnp.int32),
            pltpu.VMEM((sup,), jnp.float32),
            pltpu.VMEM((bat * 128, 16), jnp.float32),
            pltpu.VMEM((bat * 128, 16), jnp.float32),
            pltpu.VMEM_SHARED((NP, 16), jnp.float32),
            pltpu.SemaphoreType.DMA,
            pltpu.SemaphoreType.DMA,
            pltpu.SemaphoreType.DMA,
            pltpu.SemaphoreType.DMA,
        ],
    )
    def phase(*refs):
        if transform:
            (srcA_hbm, srcB_hbm, dst_hbm, w_hbm, tab_hbm, out_hbm,
             src_v, dst_v, w_v, bufA, bufB, acc,
             semA, semB, semSA, semSB) = refs
        else:
            (srcA_hbm, dst_hbm, w_hbm, tab_hbm, out_hbm,
             src_v, dst_v, w_v, bufA, bufB, acc,
             semA, semB, semSA, semSB) = refs
            srcB_hbm = srcA_hbm
        c = lax.axis_index("c")
        s = lax.axis_index("s")
        row_base = s * RPS
        zrow = jnp.zeros((16,), jnp.float32)

        @plsc.parallel_loop(0, 391, unroll=4)
        def _(i):
            bufA[i, :] = zrow

        @pl.loop(0, 16)
        def _(i):
            pltpu.sync_copy(bufA.at[pl.ds(0, 391)],
                            acc.at[pl.ds(row_base + i * 391, 391)])

        plsc.subcore_barrier()

        idx_consts = [jnp.full((16,), e, jnp.int32) for e in range(16)]
        w_id = s if transform else c * 16 + s
        wbase = w_id * (nsup * sup)

        def copy(boff, k, buf, sem):
            return pltpu.make_async_copy(
                tab_hbm.at[src_v.at[pl.ds((boff + k) * 128, 128)]],
                buf.at[pl.ds(k * 128, 128)], sem)

        def fire(boff, buf, sem):
            for k in range(bat):
                copy(boff, k, buf, sem).start()

        def scat(boff, k, buf, sem):
            return pltpu.make_async_copy(
                buf.at[pl.ds(k * 128, 128)], acc.at[dst_v.at[boff + k]], sem)

        def drain_process(boff, buf, sem, sem_s):
            for k in range(bat):
                copy(boff, k, buf, sem).wait()
            for k in range(bat):
                @plsc.parallel_loop(0, 8, unroll=2)
                def _(g):
                    wv = w_v[pl.ds((boff + k) * 128 + g * 16, 16)]
                    for e in range(16):
                        r = k * 128 + g * 16 + e
                        splat = wv.at[idx_consts[e]].get(
                            mode="promise_in_bounds")
                        buf[r, :] = buf[r, :] * splat
                scat(boff, k, buf, sem_s).start(add=True)

        def drain_scat(boff, buf, sem_s):
            for k in range(bat):
                scat(boff, k, buf, sem_s).wait()

        @pl.loop(0, nsup)
        def _(sup_i):
            off = wbase + sup_i * sup
            co = off // 128
            if transform:
                @pl.when(c == 0)
                def _():
                    pltpu.sync_copy(srcA_hbm.at[pl.ds(off, sup)], src_v)

                @pl.when(c == 1)
                def _():
                    pltpu.sync_copy(srcB_hbm.at[pl.ds(off, sup)], src_v)
            else:
                pltpu.sync_copy(srcA_hbm.at[pl.ds(off, sup)], src_v)
            pltpu.sync_copy(dst_hbm.at[pl.ds(co, nch)], dst_v)
            pltpu.sync_copy(w_hbm.at[pl.ds(off, sup)], w_v)

            fire(0, bufA, semA)

            @pl.loop(0, npair)
            def _(p):
                b0 = p * (2 * bat)
                @pl.when(p > 0)
                def _():
                    drain_scat(b0 - bat, bufB, semSB)
                fire(b0 + bat, bufB, semB)
                drain_process(b0, bufA, semA, semSA)

                drain_scat(b0, bufA, semSA)

                @pl.when(p < npair - 1)
                def _():
                    fire(b0 + 2 * bat, bufA, semA)

                drain_process(b0 + bat, bufB, semB, semSB)

            drain_scat((npair * 2 - 1) * bat, bufB, semSB)

        plsc.subcore_barrier()
        pltpu.sync_copy(acc.at[pl.ds(row_base, RPS)],
                        out_hbm.at[c, pl.ds(row_base, RPS)])

    return phase


_phase1 = _make_phase(sup=4096, nsup=25, bat=4, transform=True)
_phase2 = _make_phase(sup=2560, nsup=20, bat=5, transform=False)


def _relu_matmul(h1p, W2):
    rows = NP * 16 // 128
    blk = rows // 4
    p = h1p.reshape(2, rows, 128)
    eye8 = jnp.eye(8, dtype=jnp.float32)
    bd_a = jnp.kron(eye8, W2[:16, :])
    bd_b = jnp.kron(eye8, W2[16:, :])

    def body(p_ref, wa_ref, wb_ref, o_ref):
        ha = jnp.maximum(p_ref[0], 0.0)
        hb = jnp.maximum(p_ref[1], 0.0)
        o_ref[...] = (
            jnp.dot(ha, wa_ref[...], preferred_element_type=jnp.float32,
                    precision=lax.Precision.HIGHEST)
            + jnp.dot(hb, wb_ref[...], preferred_element_type=jnp.float32,
                      precision=lax.Precision.HIGHEST)
        )

    out = pl.pallas_call(
        body,
        grid=(4,),
        in_specs=[
            pl.BlockSpec((2, blk, 128), lambda i: (0, i, 0)),
            pl.BlockSpec((128, 128), lambda i: (0, 0)),
            pl.BlockSpec((128, 128), lambda i: (0, 0)),
        ],
        out_specs=pl.BlockSpec((blk, 128), lambda i: (i, 0)),
        out_shape=jax.ShapeDtypeStruct((rows, 128), jnp.float32),
    )(p, bd_a, bd_b)
    return out.reshape(NP, 16)


def _add_parts(parts):
    rows = NP * 16 // 128
    blk = rows // 4
    p = parts.reshape(2, rows, 128)

    def body(p_ref, o_ref):
        o_ref[...] = p_ref[0] + p_ref[1]

    out = pl.pallas_call(
        body,
        grid=(4,),
        in_specs=[pl.BlockSpec((2, blk, 128), lambda i: (0, i, 0))],
        out_specs=pl.BlockSpec((blk, 128), lambda i: (i, 0)),
        out_shape=jax.ShapeDtypeStruct((rows, 128), jnp.float32),
    )(p)
    return out.reshape(NP, 16)


def _edge_prep(edge_index, edge_weight):
    rb = 1600
    eb = rb * 128
    nr = EPAD // 128

    def body(ei_ref, w_ref, o2a, o2b, osr, odr, owr):
        i = pl.program_id(0)
        s = ei_ref[0, :].reshape(rb, 128)
        d = ei_ref[1, :].reshape(rb, 128)
        w = w_ref[...]
        io = (jax.lax.broadcasted_iota(jnp.int32, (rb, 128), 0) * 128
              + jax.lax.broadcasted_iota(jnp.int32, (rb, 128), 1) + i * eb)
        padv = (io * 977) % N
        valid = io < E
        sv = jnp.where(valid, s, padv)
        dv = jnp.where(valid, d, padv)
        wv = jnp.where(valid, w, jnp.float32(0.0))
        o2a[...] = sv + sv
        o2b[...] = sv + sv + 1
        osr[...] = sv
        odr[...] = dv
        owr[...] = wv

    i32out = jax.ShapeDtypeStruct((nr, 128), jnp.int32)
    return pl.pallas_call(
        body,
        grid=(EPAD // eb,),
        in_specs=[
            pl.BlockSpec((2, eb), lambda i: (0, i)),
            pl.BlockSpec((rb, 128), lambda i: (i, 0)),
        ],
        out_specs=[pl.BlockSpec((rb, 128), lambda i: (i, 0))] * 5,
        out_shape=[i32out, i32out, i32out, i32out,
                   jax.ShapeDtypeStruct((nr, 128), jnp.float32)],
    )(edge_index, edge_weight.reshape(E // 128, 128))


def kernel(edge_index, edge_weight, W1, W2):
    s2a, s2b, osr, odr, owr = _edge_prep(edge_index, edge_weight)
    s2a = s2a.reshape(EPAD)
    s2b = s2b.reshape(EPAD)
    osr = osr.reshape(EPAD)
    owr = owr.reshape(EPAD)
    w1r = W1.reshape(2 * N, 16)

    h1p = _phase1(s2a, s2b, odr, owr, w1r)
    s2 = _relu_matmul(h1p, W2)
    outp = _phase2(osr, odr, owr, s2)
    return _add_parts(outp)[:N]


# speedup vs baseline: 33.4645x; 1.0961x over previous
import functools

import jax
import jax.numpy as jnp
from jax import lax
from jax.experimental import pallas as pl
from jax.experimental.pallas import tpu as pltpu
from jax.experimental.pallas import tpu_sc as plsc

N = 100000
E = 1600000
NP = 100096
RPS = NP // 16
ZR = 782
EPAD = 1638400
PAD = EPAD - E

_mesh = plsc.VectorSubcoreMesh(core_axis_name="c", subcore_axis_name="s")
_sc_params = pltpu.CompilerParams(use_tc_tiling_on_sc=False)


def _make_phase(sup, nsup, bat, transform):
    nch = sup // 128
    npair = nch // bat // 2

    @functools.partial(
        pl.kernel,
        out_type=jax.ShapeDtypeStruct((2, NP, 16), jnp.float32),
        mesh=_mesh,
        compiler_params=_sc_params,
        scratch_types=[
            pltpu.VMEM((sup,), jnp.int32),
            pltpu.VMEM((sup,), jnp.int32),
            pltpu.VMEM((nch, 128), jnp.int32),
            pltpu.VMEM((nch, 128), jnp.int32),
            pltpu.VMEM((sup,), jnp.float32),
            pltpu.VMEM((sup,), jnp.float32),
            pltpu.VMEM((bat * 128, 16), jnp.float32),
            pltpu.VMEM((bat * 128, 16), jnp.float32),
            pltpu.VMEM_SHARED((NP, 16), jnp.float32),
            pltpu.SemaphoreType.DMA,
            pltpu.SemaphoreType.DMA,
            pltpu.SemaphoreType.DMA,
            pltpu.SemaphoreType.DMA,
            pltpu.SemaphoreType.DMA,
            pltpu.SemaphoreType.DMA,
        ],
    )
    def phase(*refs):
        if transform:
            (srcA_hbm, srcB_hbm, dst_hbm, w_hbm, tab_hbm, out_hbm,
             src0, src1, dst0, dst1, w0, w1, bufA, bufB, acc,
             semA, semB, semSA, semSB, semT0, semT1) = refs
        else:
            (srcA_hbm, dst_hbm, w_hbm, tab_hbm, out_hbm,
             src0, src1, dst0, dst1, w0, w1, bufA, bufB, acc,
             semA, semB, semSA, semSB, semT0, semT1) = refs
            srcB_hbm = srcA_hbm
        c = lax.axis_index("c")
        s = lax.axis_index("s")
        row_base = s * RPS
        zrow = jnp.zeros((16,), jnp.float32)

        @plsc.parallel_loop(0, 391, unroll=4)
        def _(i):
            bufA[i, :] = zrow

        @pl.loop(0, 16)
        def _(i):
            pltpu.sync_copy(bufA.at[pl.ds(0, 391)],
                            acc.at[pl.ds(row_base + i * 391, 391)])

        plsc.subcore_barrier()

        idx_consts = [jnp.full((16,), e, jnp.int32) for e in range(16)]
        w_id = s if transform else c * 16 + s
        wbase = w_id * (nsup * sup)

        def stage_start(sup_i, sv, dv, wv, semT):
            off = wbase + sup_i * sup
            co = off // 128
            if transform:
                @pl.when(c == 0)
                def _():
                    pltpu.make_async_copy(
                        srcA_hbm.at[pl.ds(off, sup)], sv, semT).start()

                @pl.when(c == 1)
                def _():
                    pltpu.make_async_copy(
                        srcB_hbm.at[pl.ds(off, sup)], sv, semT).start()
            else:
                pltpu.make_async_copy(
                    srcA_hbm.at[pl.ds(off, sup)], sv, semT).start()
            pltpu.make_async_copy(dst_hbm.at[pl.ds(co, nch)], dv, semT).start()
            pltpu.make_async_copy(w_hbm.at[pl.ds(off, sup)], wv, semT).start()

        def stage_wait(sup_i, sv, dv, wv, semT):
            off = wbase + sup_i * sup
            co = off // 128
            pltpu.make_async_copy(srcA_hbm.at[pl.ds(off, sup)], sv, semT).wait()
            pltpu.make_async_copy(dst_hbm.at[pl.ds(co, nch)], dv, semT).wait()
            pltpu.make_async_copy(w_hbm.at[pl.ds(off, sup)], wv, semT).wait()

        def copy(boff, k, buf, sem, sv):
            return pltpu.make_async_copy(
                tab_hbm.at[sv.at[pl.ds((boff + k) * 128, 128)]],
                buf.at[pl.ds(k * 128, 128)], sem)

        def scat(boff, k, buf, sem, dv):
            return pltpu.make_async_copy(
                buf.at[pl.ds(k * 128, 128)], acc.at[dv.at[boff + k]], sem)

        def process_super(sv, dv, wv):
            def fire(boff, buf, sem):
                for k in range(bat):
                    copy(boff, k, buf, sem, sv).start()

            def drain_process(boff, buf, sem, sem_s):
                for k in range(bat):
                    copy(boff, k, buf, sem, sv).wait()
                for k in range(bat):
                    @plsc.parallel_loop(0, 8, unroll=2)
                    def _(g):
                        wvec = wv[pl.ds((boff + k) * 128 + g * 16, 16)]
                        for e in range(16):
                            r = k * 128 + g * 16 + e
                            splat = wvec.at[idx_consts[e]].get(
                                mode="promise_in_bounds")
                            buf[r, :] = buf[r, :] * splat
                    scat(boff, k, buf, sem_s, dv).start(add=True)

            def drain_scat(boff, buf, sem_s):
                for k in range(bat):
                    scat(boff, k, buf, sem_s, dv).wait()

            fire(0, bufA, semA)

            @pl.loop(0, npair)
            def _(p):
                b0 = p * (2 * bat)

                @pl.when(p > 0)
                def _():
                    drain_scat(b0 - bat, bufB, semSB)

                fire(b0 + bat, bufB, semB)
                drain_process(b0, bufA, semA, semSA)
                drain_scat(b0, bufA, semSA)

                @pl.when(p < npair - 1)
                def _():
                    fire(b0 + 2 * bat, bufA, semA)

                drain_process(b0 + bat, bufB, semB, semSB)

            drain_scat((npair * 2 - 1) * bat, bufB, semSB)

        stage_start(0, src0, dst0, w0, semT0)
        nhalf = (nsup + 1) // 2

        @pl.loop(0, nhalf)
        def _(q):
            i0 = q * 2
            stage_wait(i0, src0, dst0, w0, semT0)

            @pl.when(i0 + 1 < nsup)
            def _():
                stage_start(i0 + 1, src1, dst1, w1, semT1)

            process_super(src0, dst0, w0)

            @pl.when(i0 + 1 < nsup)
            def _():
                stage_wait(i0 + 1, src1, dst1, w1, semT1)

                @pl.when(i0 + 2 < nsup)
                def _():
                    stage_start(i0 + 2, src0, dst0, w0, semT0)

                process_super(src1, dst1, w1)

        plsc.subcore_barrier()
        pltpu.sync_copy(acc.at[pl.ds(row_base, RPS)],
                        out_hbm.at[c, pl.ds(row_base, RPS)])

    return phase


_phase1 = _make_phase(sup=2048, nsup=50, bat=4, transform=True)
_phase2 = _make_phase(sup=2048, nsup=25, bat=4, transform=False)


def _relu_matmul(h1p, W2):
    rows = NP * 16 // 128
    blk = rows // 4
    p = h1p.reshape(2, rows, 128)
    eye8 = jnp.eye(8, dtype=jnp.float32)
    bd_a = jnp.kron(eye8, W2[:16, :])
    bd_b = jnp.kron(eye8, W2[16:, :])

    def body(p_ref, wa_ref, wb_ref, o_ref):
        ha = jnp.maximum(p_ref[0], 0.0)
        hb = jnp.maximum(p_ref[1], 0.0)
        o_ref[...] = (
            jnp.dot(ha, wa_ref[...], preferred_element_type=jnp.float32,
                    precision=lax.Precision.HIGHEST)
            + jnp.dot(hb, wb_ref[...], preferred_element_type=jnp.float32,
                      precision=lax.Precision.HIGHEST)
        )

    out = pl.pallas_call(
        body,
        grid=(4,),
        in_specs=[
            pl.BlockSpec((2, blk, 128), lambda i: (0, i, 0)),
            pl.BlockSpec((128, 128), lambda i: (0, 0)),
            pl.BlockSpec((128, 128), lambda i: (0, 0)),
        ],
        out_specs=pl.BlockSpec((blk, 128), lambda i: (i, 0)),
        out_shape=jax.ShapeDtypeStruct((rows, 128), jnp.float32),
    )(p, bd_a, bd_b)
    return out.reshape(NP, 16)


def _add_parts(parts):
    rows = NP * 16 // 128
    blk = rows // 4
    p = parts.reshape(2, rows, 128)

    def body(p_ref, o_ref):
        o_ref[...] = p_ref[0] + p_ref[1]

    out = pl.pallas_call(
        body,
        grid=(4,),
        in_specs=[pl.BlockSpec((2, blk, 128), lambda i: (0, i, 0))],
        out_specs=pl.BlockSpec((blk, 128), lambda i: (i, 0)),
        out_shape=jax.ShapeDtypeStruct((rows, 128), jnp.float32),
    )(p)
    return out.reshape(NP, 16)


def _edge_prep(edge_index, edge_weight):
    rb = 1600
    eb = rb * 128
    nr = EPAD // 128

    def body(ei_ref, w_ref, o2a, o2b, osr, odr, owr):
        i = pl.program_id(0)
        s = ei_ref[0, :].reshape(rb, 128)
        d = ei_ref[1, :].reshape(rb, 128)
        w = w_ref[...]
        io = (jax.lax.broadcasted_iota(jnp.int32, (rb, 128), 0) * 128
              + jax.lax.broadcasted_iota(jnp.int32, (rb, 128), 1) + i * eb)
        padv = (io * 977) % N
        valid = io < E
        sv = jnp.where(valid, s, padv)
        dv = jnp.where(valid, d, padv)
        wv = jnp.where(valid, w, jnp.float32(0.0))
        o2a[...] = sv + sv
        o2b[...] = sv + sv + 1
        osr[...] = sv
        odr[...] = dv
        owr[...] = wv

    i32out = jax.ShapeDtypeStruct((nr, 128), jnp.int32)
    return pl.pallas_call(
        body,
        grid=(EPAD // eb,),
        in_specs=[
            pl.BlockSpec((2, eb), lambda i: (0, i)),
            pl.BlockSpec((rb, 128), lambda i: (i, 0)),
        ],
        out_specs=[pl.BlockSpec((rb, 128), lambda i: (i, 0))] * 5,
        out_shape=[i32out, i32out, i32out, i32out,
                   jax.ShapeDtypeStruct((nr, 128), jnp.float32)],
    )(edge_index, edge_weight.reshape(E // 128, 128))


def kernel(edge_index, edge_weight, W1, W2):
    s2a, s2b, osr, odr, owr = _edge_prep(edge_index, edge_weight)
    s2a = s2a.reshape(EPAD)
    s2b = s2b.reshape(EPAD)
    osr = osr.reshape(EPAD)
    owr = owr.reshape(EPAD)
    w1r = W1.reshape(2 * N, 16)

    h1p = _phase1(s2a, s2b, odr, owr, w1r)
    s2 = _relu_matmul(h1p, W2)
    outp = _phase2(osr, odr, owr, s2)
    return _add_parts(outp)[:N]
